# baseline (speedup 1.0000x reference)
import jax
import jax.numpy as jnp
from jax.experimental import pallas as pl
from jax.experimental.pallas import tpu as pltpu

_D = 128
_BLK = 5000


def _gram_loss_kernel(a_ref, b_ref, out_ref, acc_ref):
    i = pl.program_id(0)

    @pl.when(i == 0)
    def _init():
        acc_ref[...] = jnp.zeros_like(acc_ref)

    a16 = a_ref[...].astype(jnp.bfloat16)
    b16 = b_ref[...].astype(jnp.bfloat16)
    d = a16.shape[1]
    ones_col = jnp.ones((d, 1), jnp.bfloat16)
    s1 = jax.lax.dot_general(
        a16 * a16, ones_col, (((1,), (0,)), ((), ())),
        preferred_element_type=jnp.float32,
    )
    s2 = jax.lax.dot_general(
        b16 * b16, ones_col, (((1,), (0,)), ((), ())),
        preferred_element_type=jnp.float32,
    )
    scale = jax.lax.rsqrt(s1 * s2 + 1e-24).astype(jnp.bfloat16)
    a_s = a16 * scale
    acc_ref[...] += jax.lax.dot_general(
        a_s, b16, (((0,), (0,)), ((), ())), preferred_element_type=jnp.float32
    )

    @pl.when(i == pl.num_programs(0) - 1)
    def _fin():
        m = acc_ref[...]
        out_ref[...] = (jnp.sum(m * m) / float(m.shape[0] * m.shape[1])).reshape(
            1, 1
        )


def kernel(input1, input2):
    n = input1.shape[0]
    a = input1.reshape(n, -1).astype(jnp.float32)
    b = input2.reshape(n, -1).astype(jnp.float32)
    d = a.shape[1]

    blk = _BLK if n % _BLK == 0 and _BLK <= n else None
    if blk is None:
        blk = min(n, _BLK)
        pad = (-n) % blk
        if pad:
            a = jnp.pad(a, ((0, pad), (0, 0)))
            b = jnp.pad(b, ((0, pad), (0, 0)))
    n_padded = a.shape[0]
    grid = n_padded // blk

    out = pl.pallas_call(
        _gram_loss_kernel,
        grid=(grid,),
        in_specs=[
            pl.BlockSpec((blk, d), lambda i: (i, 0)),
            pl.BlockSpec((blk, d), lambda i: (i, 0)),
        ],
        out_specs=pl.BlockSpec((1, 1), lambda i: (0, 0)),
        out_shape=jax.ShapeDtypeStruct((1, 1), jnp.float32),
        scratch_shapes=[pltpu.VMEM((d, d), jnp.float32)],
        compiler_params=pltpu.CompilerParams(
            dimension_semantics=("arbitrary",)
        ),
    )(a, b)
    return out[0, 0]


# speedup vs baseline: 1.2615x; 1.2615x over previous
import jax
import jax.numpy as jnp
from jax.experimental import pallas as pl
from jax.experimental.pallas import tpu as pltpu

_D = 128
_BLK = 20000


def _gram_loss_kernel(a_ref, b_ref, out_ref, acc_ref):
    i = pl.program_id(0)

    @pl.when(i == 0)
    def _init():
        acc_ref[...] = jnp.zeros_like(acc_ref)

    a16 = a_ref[...].astype(jnp.bfloat16)
    b16 = b_ref[...].astype(jnp.bfloat16)
    d = a16.shape[1]
    ones_col = jnp.ones((d, 1), jnp.bfloat16)
    s1 = jax.lax.dot_general(
        a16 * a16, ones_col, (((1,), (0,)), ((), ())),
        preferred_element_type=jnp.float32,
    )
    s2 = jax.lax.dot_general(
        b16 * b16, ones_col, (((1,), (0,)), ((), ())),
        preferred_element_type=jnp.float32,
    )
    scale = jax.lax.rsqrt(s1 * s2 + 1e-24).astype(jnp.bfloat16)
    a_s = a16 * scale
    acc_ref[...] += jax.lax.dot_general(
        a_s, b16, (((0,), (0,)), ((), ())), preferred_element_type=jnp.float32
    )

    @pl.when(i == pl.num_programs(0) - 1)
    def _fin():
        m = acc_ref[...]
        out_ref[...] = (jnp.sum(m * m) / float(m.shape[0] * m.shape[1])).reshape(
            1, 1
        )


def kernel(input1, input2):
    n = input1.shape[0]
    a = input1.reshape(n, -1).astype(jnp.float32)
    b = input2.reshape(n, -1).astype(jnp.float32)
    d = a.shape[1]

    blk = _BLK if n % _BLK == 0 and _BLK <= n else None
    if blk is None:
        blk = min(n, _BLK)
        pad = (-n) % blk
        if pad:
            a = jnp.pad(a, ((0, pad), (0, 0)))
            b = jnp.pad(b, ((0, pad), (0, 0)))
    n_padded = a.shape[0]
    grid = n_padded // blk

    out = pl.pallas_call(
        _gram_loss_kernel,
        grid=(grid,),
        in_specs=[
            pl.BlockSpec((blk, d), lambda i: (i, 0)),
            pl.BlockSpec((blk, d), lambda i: (i, 0)),
        ],
        out_specs=pl.BlockSpec((1, 1), lambda i: (0, 0)),
        out_shape=jax.ShapeDtypeStruct((1, 1), jnp.float32),
        scratch_shapes=[pltpu.VMEM((d, d), jnp.float32)],
        compiler_params=pltpu.CompilerParams(
            dimension_semantics=("arbitrary",)
        ),
    )(a, b)
    return out[0, 0]
